# Initial kernel scaffold; baseline (speedup 1.0000x reference)
import jax
import jax.numpy as jnp
from jax.experimental import pallas as pl


def kernel(x, edge_index, batch, W_phi, b_phi, W1, b1, W2, b2):
    raise NotImplementedError("write your pallas kernel here")


# speedup vs baseline: 22.8433x; 22.8433x over previous
import functools

import jax
import jax.numpy as jnp
from jax import lax
from jax.experimental import pallas as pl
from jax.experimental.pallas import tpu as pltpu
from jax.experimental.pallas import tpu_sc as plsc

_N_NODES = 10000
_N_EDGES = 320000
_D = 128
_N_GRAPHS = 64
_N_CLASSES = 10

_NC = 2
_NS = 16
_NW = _NC * _NS
_EPW = _N_EDGES // _NW
_CHUNKS = _EPW // 16
_IDXW = ((_EPW + 127) // 128) * 128
_COFF = 10048
_ACC = 655360
_APT = _ACC // _NS
_ZB = _APT // 2


def _sc_histogram(row, col, batch, ones, zeros):
    mesh = plsc.VectorSubcoreMesh(
        core_axis_name="c", subcore_axis_name="s",
        num_cores=_NC, num_subcores=_NS)

    @functools.partial(
        pl.kernel,
        out_type=jax.ShapeDtypeStruct((_NC * _ACC,), jnp.float32),
        mesh=mesh,
        scratch_types=[
            pltpu.VMEM((_EPW,), jnp.int32),
            pltpu.VMEM((_EPW,), jnp.int32),
            pltpu.VMEM((_EPW,), jnp.int32),
            pltpu.VMEM((_IDXW,), jnp.int32),
            pltpu.VMEM((_IDXW,), jnp.float32),
            pltpu.VMEM((_ZB,), jnp.float32),
            pltpu.VMEM_SHARED((_ACC,), jnp.float32),
        ],
    )
    def hist(row_hbm, col_hbm, batch_hbm, ones_hbm, zeros_hbm, out_hbm,
             row_v, col_v, g_v, idx_v, ones_v, zbuf, shared):
        cid = lax.axis_index("c")
        sid = lax.axis_index("s")
        wid = sid * _NC + cid
        base = wid * _EPW

        pltpu.sync_copy(row_hbm.at[pl.ds(base, _EPW)], row_v)
        pltpu.sync_copy(col_hbm.at[pl.ds(base, _EPW)], col_v)
        pltpu.sync_copy(batch_hbm.at[row_v], g_v)
        pltpu.sync_copy(ones_hbm, ones_v)
        pltpu.sync_copy(zeros_hbm, zbuf)
        for k in range(2):
            pltpu.sync_copy(
                zbuf, shared.at[pl.ds(sid * _APT + k * _ZB, _ZB)])

        zero16 = jnp.zeros((16,), jnp.int32)
        for k in range((_IDXW - _EPW) // 16):
            idx_v[pl.ds(_EPW + k * 16, 16)] = zero16

        def step(i, carry):
            e = i * 16
            c16 = col_v[pl.ds(e, 16)]
            g16 = g_v[pl.ds(e, 16)]
            idx_v[pl.ds(e, 16)] = g16 * _N_NODES + c16 + _COFF
            return carry

        lax.fori_loop(0, _CHUNKS, step, 0)

        plsc.subcore_barrier()
        pltpu.sync_copy(ones_v.at[pl.ds(0, _EPW)], shared.at[row_v], add=True)
        pltpu.sync_copy(ones_v, shared.at[idx_v], add=True)
        plsc.subcore_barrier()
        for k in range(2):
            pltpu.sync_copy(
                shared.at[pl.ds(sid * _APT + k * _ZB, _ZB)], zbuf)
            pltpu.sync_copy(
                zbuf, out_hbm.at[pl.ds(cid * _ACC + sid * _APT + k * _ZB,
                                       _ZB)])

    return hist(row, col, batch, ones, zeros)


def _tc_body(deg_ref, ccol_ref, batch_ref, x_ref, wphi_ref, bphi_ref,
             w1_ref, b1_ref, w2_ref, b2_ref, out_ref):
    deg = deg_ref[0] + deg_ref[1]
    ccol = ccol_ref[0] + ccol_ref[1]
    gids = lax.broadcasted_iota(jnp.int32, (_N_GRAPHS, _N_NODES), 0)
    crow = jnp.where(gids == batch_ref[...], deg, 0.0)
    P = jnp.concatenate([crow, ccol], axis=0)
    Y = lax.dot_general(
        P, x_ref[...], (((1,), (0,)), ((), ())),
        precision=lax.Precision.HIGHEST,
        preferred_element_type=jnp.float32)
    cnt = jnp.sum(crow, axis=1, keepdims=True)
    pooled = (
        jnp.dot(Y[:_N_GRAPHS], wphi_ref[:_D, :],
                precision=lax.Precision.HIGHEST)
        + jnp.dot(Y[_N_GRAPHS:], wphi_ref[_D:, :],
                  precision=lax.Precision.HIGHEST)
        + cnt * bphi_ref[...])
    h = jnp.maximum(
        jnp.dot(pooled, w1_ref[...], precision=lax.Precision.HIGHEST)
        + b1_ref[...], 0.0)
    out_ref[...] = (
        jnp.dot(h, w2_ref[...], precision=lax.Precision.HIGHEST)
        + b2_ref[...])


def kernel(x, edge_index, batch, W_phi, b_phi, W1, b1, W2, b2):
    row = edge_index[0]
    col = edge_index[1]
    ones = (jnp.arange(_IDXW, dtype=jnp.int32) < _EPW).astype(jnp.float32)
    zeros = jnp.zeros((_ZB,), jnp.float32)

    flat = _sc_histogram(row, col, batch, ones, zeros)
    percore = flat.reshape(_NC, _ACC)
    deg = percore[:, :_N_NODES].reshape(_NC, 1, _N_NODES)
    ccol = percore[:, _COFF:_COFF + _N_GRAPHS * _N_NODES].reshape(
        _NC, _N_GRAPHS, _N_NODES)

    out = pl.pallas_call(
        _tc_body,
        out_shape=jax.ShapeDtypeStruct((_N_GRAPHS, _N_CLASSES), jnp.float32),
    )(deg, ccol, batch.reshape(1, _N_NODES), x, W_phi,
      b_phi.reshape(1, _D), W1, b1.reshape(1, _D), W2,
      b2.reshape(1, _N_CLASSES))
    return out


# speedup vs baseline: 28.7574x; 1.2589x over previous
import functools

import jax
import jax.numpy as jnp
from jax import lax
from jax.experimental import pallas as pl
from jax.experimental.pallas import tpu as pltpu
from jax.experimental.pallas import tpu_sc as plsc

_N_NODES = 10000
_N_EDGES = 320000
_D = 128
_N_GRAPHS = 64
_N_CLASSES = 10

_NC = 2
_NS = 16
_NW = _NC * _NS
_EPW = _N_EDGES // _NW
_CHUNKS = _EPW // 16
_IDXW = ((_EPW + 127) // 128) * 128
_COFF = 10048
_ACC = 655360
_APT = _ACC // _NS
_ZB = _APT // 2


_CPT = (_N_GRAPHS * _N_NODES) // _NS


def _sc_histogram(edge_index, batch, ones, zeros):
    mesh = plsc.VectorSubcoreMesh(
        core_axis_name="c", subcore_axis_name="s",
        num_cores=_NC, num_subcores=_NS)

    @functools.partial(
        pl.kernel,
        out_type=(
            jax.ShapeDtypeStruct((_NC * _N_NODES,), jnp.float32),
            jax.ShapeDtypeStruct((_NC * _N_GRAPHS * _N_NODES,), jnp.float32),
        ),
        mesh=mesh,
        scratch_types=[
            pltpu.VMEM((_EPW,), jnp.int32),
            pltpu.VMEM((_EPW,), jnp.int32),
            pltpu.VMEM((_EPW,), jnp.int32),
            pltpu.VMEM((_IDXW,), jnp.int32),
            pltpu.VMEM((_IDXW,), jnp.float32),
            pltpu.VMEM((_ZB,), jnp.float32),
            pltpu.VMEM_SHARED((_ACC,), jnp.float32),
        ],
    )
    def hist(edge_hbm, batch_hbm, ones_hbm, zeros_hbm, deg_hbm, ccol_hbm,
             row_v, col_v, g_v, idx_v, ones_v, zbuf, shared):
        cid = lax.axis_index("c")
        sid = lax.axis_index("s")
        wid = sid * _NC + cid
        base = wid * _EPW

        pltpu.sync_copy(edge_hbm.at[pl.ds(base, _EPW)], row_v)
        pltpu.sync_copy(edge_hbm.at[pl.ds(_N_EDGES + base, _EPW)], col_v)
        pltpu.sync_copy(batch_hbm.at[row_v], g_v)
        pltpu.sync_copy(ones_hbm, ones_v)
        pltpu.sync_copy(zeros_hbm, zbuf)
        for k in range(2):
            pltpu.sync_copy(
                zbuf, shared.at[pl.ds(sid * _APT + k * _ZB, _ZB)])

        zero16 = jnp.zeros((16,), jnp.int32)
        for k in range((_IDXW - _EPW) // 16):
            idx_v[pl.ds(_EPW + k * 16, 16)] = zero16

        def step(i, carry):
            e = i * 16
            c16 = col_v[pl.ds(e, 16)]
            g16 = g_v[pl.ds(e, 16)]
            idx_v[pl.ds(e, 16)] = g16 * _N_NODES + c16 + _COFF
            return carry

        lax.fori_loop(0, _CHUNKS, step, 0)

        plsc.subcore_barrier()
        pltpu.sync_copy(ones_v.at[pl.ds(0, _EPW)], shared.at[row_v], add=True)
        pltpu.sync_copy(ones_v, shared.at[idx_v], add=True)
        plsc.subcore_barrier()
        @pl.when(sid == 0)
        def _():
            pltpu.sync_copy(shared.at[pl.ds(0, _N_NODES)],
                            zbuf.at[pl.ds(0, _N_NODES)])
            pltpu.sync_copy(zbuf.at[pl.ds(0, _N_NODES)],
                            deg_hbm.at[pl.ds(cid * _N_NODES, _N_NODES)])

        for k in range(2):
            half = _CPT // 2
            pltpu.sync_copy(
                shared.at[pl.ds(_COFF + sid * _CPT + k * half, half)],
                zbuf.at[pl.ds(0, half)])
            pltpu.sync_copy(
                zbuf.at[pl.ds(0, half)],
                ccol_hbm.at[pl.ds(cid * _N_GRAPHS * _N_NODES + sid * _CPT
                                  + k * half, half)])

    return hist(edge_index, batch, ones, zeros)


def _tc_body(deg_ref, ccol_ref, batch_ref, x_ref, wphi_ref, bphi_ref,
             w1_ref, b1_ref, w2_ref, b2_ref, out_ref):
    deg = deg_ref[0] + deg_ref[1]
    ccol = ccol_ref[0] + ccol_ref[1]
    gids = lax.broadcasted_iota(jnp.int32, (_N_GRAPHS, _N_NODES), 0)
    crow = jnp.where(gids == batch_ref[...], deg, 0.0)
    P = jnp.concatenate([crow, ccol], axis=0)
    Y = lax.dot_general(
        P, x_ref[...], (((1,), (0,)), ((), ())),
        precision=lax.Precision.HIGHEST,
        preferred_element_type=jnp.float32)
    cnt = jnp.sum(crow, axis=1, keepdims=True)
    pooled = (
        jnp.dot(Y[:_N_GRAPHS], wphi_ref[:_D, :],
                precision=lax.Precision.HIGHEST)
        + jnp.dot(Y[_N_GRAPHS:], wphi_ref[_D:, :],
                  precision=lax.Precision.HIGHEST)
        + cnt * bphi_ref[...])
    h = jnp.maximum(
        jnp.dot(pooled, w1_ref[...], precision=lax.Precision.HIGHEST)
        + b1_ref[...], 0.0)
    out_ref[...] = (
        jnp.dot(h, w2_ref[...], precision=lax.Precision.HIGHEST)
        + b2_ref[...])


def kernel(x, edge_index, batch, W_phi, b_phi, W1, b1, W2, b2):
    ones = (jnp.arange(_IDXW, dtype=jnp.int32) < _EPW).astype(jnp.float32)
    zeros = jnp.zeros((_ZB,), jnp.float32)

    deg_rw, ccol_rw = _sc_histogram(edge_index.reshape(2 * _N_EDGES), batch,
                                    ones, zeros)
    deg = deg_rw.reshape(_NC, 1, _N_NODES)
    ccol = ccol_rw.reshape(_NC, _N_GRAPHS, _N_NODES)

    out = pl.pallas_call(
        _tc_body,
        out_shape=jax.ShapeDtypeStruct((_N_GRAPHS, _N_CLASSES), jnp.float32),
    )(deg, ccol, batch.reshape(1, _N_NODES), x, W_phi,
      b_phi.reshape(1, _D), W1, b1.reshape(1, _D), W2,
      b2.reshape(1, _N_CLASSES))
    return out
